# Initial kernel scaffold; baseline (speedup 1.0000x reference)
import functools

import jax
import jax.numpy as jnp
from jax import lax
from jax.experimental import pallas as pl
from jax.experimental.pallas import tpu as pltpu
from jax.experimental.pallas import tpu_sc as plsc

D_ITEM = 64
D_OTHER = 16
_CH = 128


def _item_transform(item_table, wfi):
    V = item_table.shape[0]
    BR = 2048
    grid = (V + BR - 1) // BR

    def body(t_ref, w_ref, o_ref):
        o_ref[...] = lax.dot_general(t_ref[...], w_ref[...],
                                     (((1,), (1,)), ((), ())),
                                     preferred_element_type=jnp.float32)

    return pl.pallas_call(
        body,
        grid=(grid,),
        in_specs=[pl.BlockSpec((BR, D_ITEM), lambda i: (i, 0)),
                  pl.BlockSpec((D_ITEM, D_ITEM), lambda i: (0, 0))],
        out_specs=pl.BlockSpec((BR, D_ITEM), lambda i: (i, 0)),
        out_shape=jax.ShapeDtypeStruct((V, D_ITEM), jnp.float32),
    )(item_table, wfi)


def _small_transforms(brand_table, cat_table, wfb, wfc, wfp, Wp, bp2, bf2):
    def body(bt, ct, wb, wc, wpf, wpp, bpr, bfr, ob, oc, opv):
        ob[...] = lax.dot_general(bt[...], wb[...], (((1,), (1,)), ((), ())),
                                  preferred_element_type=jnp.float32)
        c = lax.dot_general(bpr[...], wpf[...], (((1,), (1,)), ((), ())),
                            preferred_element_type=jnp.float32) + bfr[...]
        oc[...] = lax.dot_general(ct[...], wc[...], (((1,), (1,)), ((), ())),
                                  preferred_element_type=jnp.float32) + c
        opv[...] = lax.dot_general(wpp[...], wpf[...], (((0,), (1,)), ((), ())),
                                   preferred_element_type=jnp.float32)

    nb = brand_table.shape[0]
    nc = cat_table.shape[0]
    return pl.pallas_call(
        body,
        out_shape=[jax.ShapeDtypeStruct((nb, D_ITEM), jnp.float32),
                   jax.ShapeDtypeStruct((nc, D_ITEM), jnp.float32),
                   jax.ShapeDtypeStruct((1, D_ITEM), jnp.float32)],
    )(brand_table, cat_table, wfb, wfc, wfp, Wp, bp2, bf2)


def _sc_fuse(item_idx, brand_idx, cat_idx, price, item2, brand2, cat2c, pv):
    n = item_idx.shape[0]
    info = plsc.get_sparse_core_info()
    nw = info.num_cores * info.num_subcores
    assert n % (nw * _CH) == 0
    rows_w = n // nw
    nch = rows_w // _CH
    mesh = plsc.VectorSubcoreMesh(core_axis_name="c", subcore_axis_name="s")

    @functools.partial(
        pl.kernel, mesh=mesh,
        out_type=jax.ShapeDtypeStruct((n, D_ITEM), jnp.float32),
        scratch_types=[
            pltpu.VMEM((_CH,), jnp.int32),
            pltpu.VMEM((_CH,), jnp.int32),
            pltpu.VMEM((_CH,), jnp.int32),
            pltpu.VMEM((_CH,), jnp.float32),
            pltpu.VMEM((_CH, D_ITEM), jnp.float32),
            pltpu.VMEM((_CH, D_ITEM), jnp.float32),
            pltpu.VMEM((_CH, D_ITEM), jnp.float32),
            pltpu.VMEM((D_ITEM,), jnp.float32),
            pltpu.SemaphoreType.DMA,
            pltpu.SemaphoreType.DMA,
            pltpu.SemaphoreType.DMA,
        ],
    )
    def k(ii_hbm, bi_hbm, ci_hbm, pr_hbm, it2_hbm, br2_hbm, ct2_hbm, pv_hbm,
          out_hbm, ii_v, bi_v, ci_v, pr_v, bufa, bufb, bufc, pv_v, s1, s2, s3):
        wid = lax.axis_index("s") * info.num_cores + lax.axis_index("c")
        base = wid * rows_w
        pltpu.sync_copy(pv_hbm.at[0], pv_v)

        def chunk(g, carry):
            off = base + g * _CH
            pltpu.sync_copy(ii_hbm.at[pl.ds(off, _CH)], ii_v)
            pltpu.sync_copy(bi_hbm.at[pl.ds(off, _CH)], bi_v)
            pltpu.sync_copy(ci_hbm.at[pl.ds(off, _CH)], ci_v)
            pltpu.sync_copy(pr_hbm.at[pl.ds(off, _CH)], pr_v)
            c1 = pltpu.async_copy(it2_hbm.at[ii_v], bufa, s1)
            c2 = pltpu.async_copy(br2_hbm.at[bi_v], bufb, s2)
            c3 = pltpu.async_copy(ct2_hbm.at[ci_v], bufc, s3)
            c1.wait()
            c2.wait()
            c3.wait()
            pvs = [pv_v[pl.ds(t * 16, 16)] for t in range(4)]

            def rows16(j, carry2):
                pvec = pr_v[pl.ds(j * 16, 16)]
                for r in range(16):
                    nrow = j * 16 + r
                    pb = jnp.take(pvec, jnp.full((16,), r, jnp.int32),
                                  mode="promise_in_bounds")
                    for t in range(4):
                        sl = pl.ds(t * 16, 16)
                        bufa[nrow, sl] = (bufa[nrow, sl] + bufb[nrow, sl]
                                          + bufc[nrow, sl] + pb * pvs[t])
                return carry2

            lax.fori_loop(0, _CH // 16, rows16, 0)
            pltpu.sync_copy(bufa, out_hbm.at[pl.ds(off, _CH)])
            return carry

        lax.fori_loop(0, nch, chunk, 0)

    return k(item_idx, brand_idx, cat_idx, price, item2, brand2, cat2c, pv)


def kernel(x, item_table, brand_table, cat_table, Wp, bp, Wf, bf):
    item_idx = x[:, 0].astype(jnp.int32)
    brand_idx = x[:, 1].astype(jnp.int32)
    cat_idx = x[:, 2].astype(jnp.int32)
    price = x[:, 3]
    wfi = Wf[:, :D_ITEM]
    wfb = Wf[:, D_ITEM:D_ITEM + D_OTHER]
    wfc = Wf[:, D_ITEM + D_OTHER:D_ITEM + 2 * D_OTHER]
    wfp = Wf[:, D_ITEM + 2 * D_OTHER:]
    item2 = _item_transform(item_table, wfi)
    brand2, cat2c, pv = _small_transforms(
        brand_table, cat_table, wfb, wfc, wfp, Wp,
        bp.reshape(1, -1), bf.reshape(1, -1))
    return _sc_fuse(item_idx, brand_idx, cat_idx, price, item2, brand2, cat2c, pv)


# speedup vs baseline: 3.8678x; 3.8678x over previous
import functools

import jax
import jax.numpy as jnp
from jax import lax
from jax.experimental import pallas as pl
from jax.experimental.pallas import tpu as pltpu
from jax.experimental.pallas import tpu_sc as plsc

D_ITEM = 64
D_OTHER = 16
_CH = 128


def _item_transform(item_table, wfi):
    V = item_table.shape[0]
    BR = 2048
    grid = (V + BR - 1) // BR

    def body(t_ref, w_ref, o_ref):
        o_ref[...] = lax.dot_general(t_ref[...], w_ref[...],
                                     (((1,), (1,)), ((), ())),
                                     preferred_element_type=jnp.float32)

    return pl.pallas_call(
        body,
        grid=(grid,),
        in_specs=[pl.BlockSpec((BR, D_ITEM), lambda i: (i, 0)),
                  pl.BlockSpec((D_ITEM, D_ITEM), lambda i: (0, 0))],
        out_specs=pl.BlockSpec((BR, D_ITEM), lambda i: (i, 0)),
        out_shape=jax.ShapeDtypeStruct((V, D_ITEM), jnp.float32),
    )(item_table, wfi)


def _small_transforms(brand_table, cat_table, wfb, wfc, wfp, Wp, bp2, bf2):
    def body(bt, ct, wb, wc, wpf, wpp, bpr, bfr, ob, oc, opv):
        ob[...] = lax.dot_general(bt[...], wb[...], (((1,), (1,)), ((), ())),
                                  preferred_element_type=jnp.float32)
        c = lax.dot_general(bpr[...], wpf[...], (((1,), (1,)), ((), ())),
                            preferred_element_type=jnp.float32) + bfr[...]
        oc[...] = lax.dot_general(ct[...], wc[...], (((1,), (1,)), ((), ())),
                                  preferred_element_type=jnp.float32) + c
        opv[...] = lax.dot_general(wpp[...], wpf[...], (((0,), (1,)), ((), ())),
                                   preferred_element_type=jnp.float32)

    nb = brand_table.shape[0]
    nc = cat_table.shape[0]
    return pl.pallas_call(
        body,
        out_shape=[jax.ShapeDtypeStruct((nb, D_ITEM), jnp.float32),
                   jax.ShapeDtypeStruct((nc, D_ITEM), jnp.float32),
                   jax.ShapeDtypeStruct((1, D_ITEM), jnp.float32)],
    )(brand_table, cat_table, wfb, wfc, wfp, Wp, bp2, bf2)


def _sc_fuse(item_idx, brand_idx, cat_idx, price, item2, brand2, cat2c, pv):
    n = item_idx.shape[0]
    info = plsc.get_sparse_core_info()
    nw = info.num_cores * info.num_subcores
    assert n % (nw * _CH) == 0
    rows_w = n // nw
    nch = rows_w // _CH
    mesh = plsc.VectorSubcoreMesh(core_axis_name="c", subcore_axis_name="s")

    @functools.partial(
        pl.kernel, mesh=mesh,
        compiler_params=pltpu.CompilerParams(use_tc_tiling_on_sc=False),
        out_type=jax.ShapeDtypeStruct((n, D_ITEM), jnp.float32),
        scratch_types=[
            pltpu.VMEM((_CH,), jnp.int32),
            pltpu.VMEM((_CH,), jnp.int32),
            pltpu.VMEM((_CH,), jnp.int32),
            pltpu.VMEM((_CH,), jnp.float32),
            pltpu.VMEM((_CH, D_ITEM), jnp.float32),
            pltpu.VMEM((_CH, D_ITEM), jnp.float32),
            pltpu.VMEM((_CH, D_ITEM), jnp.float32),
            pltpu.VMEM((D_ITEM,), jnp.float32),
            pltpu.SemaphoreType.DMA,
            pltpu.SemaphoreType.DMA,
            pltpu.SemaphoreType.DMA,
        ],
    )
    def k(ii_hbm, bi_hbm, ci_hbm, pr_hbm, it2_hbm, br2_hbm, ct2_hbm, pv_hbm,
          out_hbm, ii_v, bi_v, ci_v, pr_v, bufa, bufb, bufc, pv_v, s1, s2, s3):
        wid = lax.axis_index("s") * info.num_cores + lax.axis_index("c")
        base = wid * rows_w
        pltpu.sync_copy(pv_hbm.at[0], pv_v)

        def chunk(g, carry):
            off = base + g * _CH
            pltpu.sync_copy(ii_hbm.at[pl.ds(off, _CH)], ii_v)
            pltpu.sync_copy(bi_hbm.at[pl.ds(off, _CH)], bi_v)
            pltpu.sync_copy(ci_hbm.at[pl.ds(off, _CH)], ci_v)
            pltpu.sync_copy(pr_hbm.at[pl.ds(off, _CH)], pr_v)
            c1 = pltpu.async_copy(it2_hbm.at[ii_v], bufa, s1)
            c2 = pltpu.async_copy(br2_hbm.at[bi_v], bufb, s2)
            c3 = pltpu.async_copy(ct2_hbm.at[ci_v], bufc, s3)
            c1.wait()
            c2.wait()
            c3.wait()
            pvs = [pv_v[pl.ds(t * 16, 16)] for t in range(4)]

            def rows16(j, carry2):
                pvec = pr_v[pl.ds(j * 16, 16)]
                for r in range(16):
                    nrow = j * 16 + r
                    pb = pvec.at[jnp.full((16,), r, jnp.int32)].get(
                        mode="promise_in_bounds")
                    for t in range(4):
                        sl = pl.ds(t * 16, 16)
                        bufa[nrow, sl] = (bufa[nrow, sl] + bufb[nrow, sl]
                                          + bufc[nrow, sl] + pb * pvs[t])
                return carry2

            lax.fori_loop(0, _CH // 16, rows16, 0)
            pltpu.sync_copy(bufa, out_hbm.at[pl.ds(off, _CH)])
            return carry

        lax.fori_loop(0, nch, chunk, 0)

    return k(item_idx, brand_idx, cat_idx, price, item2, brand2, cat2c, pv)


def kernel(x, item_table, brand_table, cat_table, Wp, bp, Wf, bf):
    item_idx = x[:, 0].astype(jnp.int32)
    brand_idx = x[:, 1].astype(jnp.int32)
    cat_idx = x[:, 2].astype(jnp.int32)
    price = x[:, 3]
    wfi = Wf[:, :D_ITEM]
    wfb = Wf[:, D_ITEM:D_ITEM + D_OTHER]
    wfc = Wf[:, D_ITEM + D_OTHER:D_ITEM + 2 * D_OTHER]
    wfp = Wf[:, D_ITEM + 2 * D_OTHER:]
    item2 = _item_transform(item_table, wfi)
    brand2, cat2c, pv = _small_transforms(
        brand_table, cat_table, wfb, wfc, wfp, Wp,
        bp.reshape(1, -1), bf.reshape(1, -1))
    return _sc_fuse(item_idx, brand_idx, cat_idx, price, item2, brand2, cat2c, pv)
